# baseline (speedup 1.0000x reference)
import jax
import jax.numpy as jnp
from jax.experimental import pallas as pl

_N_EXPERTS = 64
_TOP_K = 8
_SUB = 16


def _sub_gate(x, wt):
    scores = jnp.dot(x, wt, preferred_element_type=jnp.float32)
    st = scores.T
    t = st.shape[1]

    b = jax.lax.bitcast_convert_type(st, jnp.int32)
    mono = b ^ jax.lax.shift_right_logical(
        jax.lax.shift_right_arithmetic(b, 31), 1)
    rev_iota = (_N_EXPERTS - 1) - jax.lax.broadcasted_iota(
        jnp.int32, (_N_EXPERTS, t), 0)
    key = (mono & jnp.int32(~(_N_EXPERTS - 1))) | rev_iota

    picks = []
    for k in range(_TOP_K):
        mk = jnp.max(key, axis=0, keepdims=True)
        picks.append(mk)
        if k < _TOP_K - 1:
            key = jnp.where(key == mk, jnp.int32(-2147483648), key)

    pk = jnp.concatenate(picks, axis=0)
    ids = (_N_EXPERTS - 1) - (pk & jnp.int32(_N_EXPERTS - 1))
    vb = pk & jnp.int32(~(_N_EXPERTS - 1))
    vb = vb ^ jax.lax.shift_right_logical(
        jax.lax.shift_right_arithmetic(vb, 31), 1)
    v = jax.lax.bitcast_convert_type(vb, jnp.float32)
    e = jnp.exp(v - v[0:1, :])
    w = e / jnp.sum(e, axis=0, keepdims=True)
    return ids.T, w.T


def _gate_kernel(x_ref, wt_ref, idx_ref, w_ref):
    wt = wt_ref[...]
    t = x_ref.shape[0] // _SUB
    for s in range(_SUB):
        ids, w = _sub_gate(x_ref[s * t:(s + 1) * t, :], wt)
        idx_ref[s * t:(s + 1) * t, :] = ids
        w_ref[s * t:(s + 1) * t, :] = w


def kernel(hidden_states, weight):
    x = hidden_states.reshape(-1, hidden_states.shape[-1])
    n, h = x.shape
    wt = weight.T
    t = 4096
    idx_t, w_t = pl.pallas_call(
        _gate_kernel,
        grid=(n // t,),
        in_specs=[
            pl.BlockSpec((t, h), lambda i: (i, 0)),
            pl.BlockSpec((h, _N_EXPERTS), lambda i: (0, 0)),
        ],
        out_specs=[
            pl.BlockSpec((t, _TOP_K), lambda i: (i, 0)),
            pl.BlockSpec((t, _TOP_K), lambda i: (i, 0)),
        ],
        out_shape=[
            jax.ShapeDtypeStruct((n, _TOP_K), jnp.int32),
            jax.ShapeDtypeStruct((n, _TOP_K), jnp.float32),
        ],
    )(x, wt)
    return idx_t, w_t


# speedup vs baseline: 1.8515x; 1.8515x over previous
import jax
import jax.numpy as jnp
from jax.experimental import pallas as pl

_N_EXPERTS = 64
_TOP_K = 8
_SUB = 16


def _sub_gate(x, wt):
    scores = jnp.dot(x, wt, preferred_element_type=jnp.float32)
    st = scores.T
    t = st.shape[1]

    b = jax.lax.bitcast_convert_type(st, jnp.int32)
    mono = b ^ jax.lax.shift_right_logical(
        jax.lax.shift_right_arithmetic(b, 31), 1)
    rev_iota = (_N_EXPERTS - 1) - jax.lax.broadcasted_iota(
        jnp.int32, (_N_EXPERTS, t), 0)
    key = (mono & jnp.int32(~(_N_EXPERTS - 1))) | rev_iota

    picks = []
    for k in range(_TOP_K):
        mk = jnp.max(key, axis=0, keepdims=True)
        picks.append(mk)
        if k < _TOP_K - 1:
            key = jnp.where(key == mk, jnp.int32(-2147483648), key)

    pk = jnp.concatenate(picks, axis=0)
    ids = (_N_EXPERTS - 1) - (pk & jnp.int32(_N_EXPERTS - 1))
    vb = pk & jnp.int32(~(_N_EXPERTS - 1))
    vb = vb ^ jax.lax.shift_right_logical(
        jax.lax.shift_right_arithmetic(vb, 31), 1)
    v = jax.lax.bitcast_convert_type(vb, jnp.float32)
    e = jnp.exp(v - v[0:1, :])
    w = e / jnp.sum(e, axis=0, keepdims=True)
    return ids, w


def _gate_kernel(x_ref, wt_ref, idx_ref, w_ref):
    wt = wt_ref[...]
    t = x_ref.shape[0] // _SUB
    for s in range(_SUB):
        ids, w = _sub_gate(x_ref[s * t:(s + 1) * t, :], wt)
        idx_ref[:, s * t:(s + 1) * t] = ids
        w_ref[:, s * t:(s + 1) * t] = w


def kernel(hidden_states, weight):
    x = hidden_states.reshape(-1, hidden_states.shape[-1])
    n, h = x.shape
    wt = weight.T
    t = 4096
    idx_t, w_t = pl.pallas_call(
        _gate_kernel,
        grid=(n // t,),
        in_specs=[
            pl.BlockSpec((t, h), lambda i: (i, 0)),
            pl.BlockSpec((h, _N_EXPERTS), lambda i: (0, 0)),
        ],
        out_specs=[
            pl.BlockSpec((_TOP_K, t), lambda i: (0, i)),
            pl.BlockSpec((_TOP_K, t), lambda i: (0, i)),
        ],
        out_shape=[
            jax.ShapeDtypeStruct((_TOP_K, n), jnp.int32),
            jax.ShapeDtypeStruct((_TOP_K, n), jnp.float32),
        ],
    )(x, wt)
    return idx_t.T, w_t.T
